# baseline (speedup 1.0000x reference)
import functools

import jax
import jax.numpy as jnp
from jax import lax
from jax.experimental import pallas as pl
from jax.experimental.pallas import tpu as pltpu
from jax.experimental.pallas import tpu_sc as plsc

VOCAB_SIZE = 1_000_000
EMBED_DIM = 64
BATCH = 4096
SEQ_LEN = 200
MAX_WAVELENGTH = 10000.0

NUM_CORES = 2
NUM_SUBCORES = 16
NW = NUM_CORES * NUM_SUBCORES
LANES = 16
VT_FULL = VOCAB_SIZE // 128
PAIR_ROWS = VOCAB_SIZE // 2
BPW = BATCH // NW
NBUF = 4


def _pos_encoding():
    position = jnp.arange(SEQ_LEN, dtype=jnp.float32)
    min_freq = 1.0 / MAX_WAVELENGTH
    timescales = jnp.power(
        min_freq,
        (2.0 * (jnp.arange(EMBED_DIM, dtype=jnp.float32) // 2)) / float(EMBED_DIM),
    )
    angles = position[:, None] * timescales[None, :]
    cos_mask = jnp.asarray(jnp.arange(EMBED_DIM) % 2, dtype=jnp.float32)
    sin_mask = 1.0 - cos_mask
    return jnp.sin(angles) * sin_mask + jnp.cos(angles) * cos_mask


_mesh = plsc.VectorSubcoreMesh(core_axis_name="c", subcore_axis_name="s")
_params = pltpu.CompilerParams(use_tc_tiling_on_sc=True, needs_layout_passes=False)


@functools.partial(
    pl.kernel,
    out_type=jax.ShapeDtypeStruct((PAIR_ROWS, 128), jnp.float32),
    mesh=_mesh,
    compiler_params=_params,
    scratch_types=[
        pltpu.VMEM((EMBED_DIM, 128), jnp.float32),
        pltpu.VMEM((EMBED_DIM, 128), jnp.float32),
        pltpu.VMEM((32, 128), jnp.float32),
        pltpu.SemaphoreType.DMA,
    ],
)
def _repack_kernel(tt_hbm, tail_hbm, t2_hbm, src_v, dst_v, tail_v, sem):
    wid = lax.axis_index("s") * NUM_CORES + lax.axis_index("c")
    iota16 = lax.iota(jnp.int32, LANES)

    def block(i, carry):
        vt = wid + NW * i

        @pl.when(vt < VT_FULL)
        def _():
            pltpu.sync_copy(tt_hbm.at[:, pl.ds(vt * 128, 128)], src_v)

            def row(p, c):
                for half in range(2):
                    vl = 2 * p + half
                    col = jnp.zeros((LANES,), jnp.int32) + vl
                    for q in range(EMBED_DIM // LANES):
                        dst_v[p, pl.ds(half * EMBED_DIM + q * LANES, LANES)] = (
                            plsc.load_gather(src_v, [iota16 + q * LANES, col])
                        )
                return c

            lax.fori_loop(0, EMBED_DIM, row, 0)
            pltpu.sync_copy(dst_v, t2_hbm.at[pl.ds(vt * EMBED_DIM, EMBED_DIM)])

        return carry

    lax.fori_loop(0, (VT_FULL + NW - 1) // NW, block, 0)

    @pl.when(wid == NW - 1)
    def _():
        pltpu.sync_copy(tail_hbm, tail_v)
        pltpu.sync_copy(tail_v, t2_hbm.at[pl.ds(VT_FULL * EMBED_DIM, 32)])


@functools.partial(
    pl.kernel,
    out_type=jax.ShapeDtypeStruct((SEQ_LEN, 8, NW, 8, 128), jnp.float32),
    mesh=_mesh,
    compiler_params=_params,
    scratch_types=(
        [pltpu.VMEM((128, 128), jnp.float32) for _ in range(NBUF)]
        + [pltpu.VMEM((128,), jnp.int32) for _ in range(NBUF)]
        + [pltpu.VMEM((8, 8, 128), jnp.float32) for _ in range(2)]
        + [
            pltpu.VMEM((SEQ_LEN, 128), jnp.int32),
            pltpu.VMEM((SEQ_LEN // 2, 128), jnp.float32),
        ]
        + [pltpu.SemaphoreType.DMA] * NBUF
        + [pltpu.SemaphoreType.DMA] * 2
    ),
)
def _emb_kernel(x_hbm, t2_hbm, pos_hbm, out_hbm, *scratch):
    rows = scratch[:NBUF]
    pidx = scratch[NBUF : 2 * NBUF]
    slab = scratch[2 * NBUF : 2 * NBUF + 2]
    idx_v = scratch[2 * NBUF + 2]
    pos_v = scratch[2 * NBUF + 3]
    sg = scratch[2 * NBUF + 4 : 2 * NBUF + 4 + NBUF]
    so = scratch[2 * NBUF + 4 + NBUF :]

    wid = lax.axis_index("s") * NUM_CORES + lax.axis_index("c")
    iota16 = lax.iota(jnp.int32, LANES)
    pltpu.sync_copy(x_hbm.at[wid], idx_v)
    pltpu.sync_copy(pos_hbm, pos_v)

    def fire_gather(l, a):
        for q in range(128 // LANES):
            sl = pl.ds(q * LANES, LANES)
            pidx[a][sl] = lax.shift_right_logical(idx_v[l, sl], 1)
        pltpu.async_copy(t2_hbm.at[pidx[a]], rows[a], sg[a])

    def wait_gather(a):
        pltpu.make_async_copy(t2_hbm.at[pidx[a]], rows[a], sg[a]).wait()

    def wait_out(a):
        pltpu.make_async_copy(slab[a], out_hbm.at[0, :, 0], so[a]).wait()

    fire_gather(0, 0)
    fire_gather(1, 1)

    def outer(ll, carry):
        for a in range(NBUF):
            l = ll * NBUF + a

            @pl.when(l >= 2)
            def _():
                wait_out(a % 2)

            wait_gather(a)

            ph = (l & 1) * EMBED_DIM
            pvec = [
                pos_v[l // 2, pl.ds(ph + q * LANES, LANES)]
                for q in range(EMBED_DIM // LANES)
            ]

            def blkfn(blk, c):
                voff = (idx_v[l, pl.ds(blk * LANES, LANES)] & 1) * EMBED_DIM
                bl = iota16 + blk * LANES
                for d8 in range(8):
                    for dd in range(8):
                        d = d8 * 8 + dd
                        vals = plsc.load_gather(rows[a], [bl, voff + d])
                        slab[a % 2][d8, dd, pl.ds(blk * LANES, LANES)] = (
                            vals + pvec[d // LANES][d % LANES]
                        )
                return c

            lax.fori_loop(0, 128 // LANES, blkfn, 0)
            pltpu.async_copy(slab[a % 2], out_hbm.at[l, :, wid], so[a % 2])

            ln = l + 2

            @pl.when(ln < SEQ_LEN)
            def _():
                fire_gather(ln, (a + 2) % NBUF)

        return carry

    lax.fori_loop(0, SEQ_LEN // NBUF, outer, 0)

    for a in range(2):
        wait_out(a)


def kernel(x, token_emb_table):
    pos2 = _pos_encoding().reshape(SEQ_LEN // 2, 128)
    tt = token_emb_table.T
    tail = token_emb_table[VT_FULL * 128 :].reshape(32, 128)
    t2 = _repack_kernel(tt, tail)
    x_t = x.astype(jnp.int32).reshape(NW, BPW, SEQ_LEN).transpose(0, 2, 1)
    out5 = _emb_kernel(x_t, t2, pos2)
    return out5.transpose(2, 4, 0, 1, 3).reshape(BATCH, SEQ_LEN, EMBED_DIM)


# speedup vs baseline: 2.6981x; 2.6981x over previous
import functools

import jax
import jax.numpy as jnp
from jax import lax
from jax.experimental import pallas as pl
from jax.experimental.pallas import tpu as pltpu
from jax.experimental.pallas import tpu_sc as plsc

VOCAB_SIZE = 1_000_000
EMBED_DIM = 64
BATCH = 4096
SEQ_LEN = 200
MAX_WAVELENGTH = 10000.0

NUM_CORES = 2
NUM_SUBCORES = 16
NW = NUM_CORES * NUM_SUBCORES
TPW = BATCH * SEQ_LEN // NW
CHUNK = 128
NCHUNK = TPW // CHUNK
NBUF = 4
LANES = 16


def _pos_encoding():
    position = jnp.arange(SEQ_LEN, dtype=jnp.float32)
    min_freq = 1.0 / MAX_WAVELENGTH
    timescales = jnp.power(
        min_freq,
        (2.0 * (jnp.arange(EMBED_DIM, dtype=jnp.float32) // 2)) / float(EMBED_DIM),
    )
    angles = position[:, None] * timescales[None, :]
    cos_mask = jnp.asarray(jnp.arange(EMBED_DIM) % 2, dtype=jnp.float32)
    sin_mask = 1.0 - cos_mask
    return jnp.sin(angles) * sin_mask + jnp.cos(angles) * cos_mask


_mesh = plsc.VectorSubcoreMesh(core_axis_name="c", subcore_axis_name="s")
_params = pltpu.CompilerParams(use_tc_tiling_on_sc=True, needs_layout_passes=False)


@functools.partial(
    pl.kernel,
    out_type=jax.ShapeDtypeStruct((BATCH * SEQ_LEN, 128), jnp.float32),
    mesh=_mesh,
    compiler_params=_params,
    scratch_types=(
        [pltpu.VMEM((CHUNK, 128), jnp.float32) for _ in range(NBUF)]
        + [
            pltpu.VMEM((NCHUNK, CHUNK), jnp.int32),
            pltpu.VMEM((SEQ_LEN // 2, 128), jnp.float32),
        ]
        + [pltpu.SemaphoreType.DMA] * NBUF
        + [pltpu.SemaphoreType.DMA] * NBUF
    ),
)
def _emb_kernel(x_hbm, table_hbm, pos_hbm, out_hbm, *scratch):
    rows = scratch[:NBUF]
    idx_v = scratch[NBUF]
    pos_v = scratch[NBUF + 1]
    sg = scratch[NBUF + 2 : NBUF + 2 + NBUF]
    so = scratch[NBUF + 2 + NBUF :]

    wid = lax.axis_index("s") * NUM_CORES + lax.axis_index("c")
    base = wid * TPW
    pltpu.sync_copy(x_hbm.at[wid], idx_v)
    pltpu.sync_copy(pos_hbm, pos_v)

    def fire_gather(t, a):
        pltpu.async_copy(table_hbm.at[idx_v.at[t]], rows[a], sg[a])

    def wait_gather(t, a):
        pltpu.make_async_copy(table_hbm.at[idx_v.at[t]], rows[a], sg[a]).wait()

    def wait_out(a):
        pltpu.make_async_copy(rows[a], out_hbm.at[pl.ds(base, CHUNK)], so[a]).wait()

    fire_gather(0, 0)
    fire_gather(1, 1)

    def outer(tt, carry):
        for a in range(NBUF):
            t = tt * NBUF + a

            @pl.when(t >= NBUF)
            def _():
                wait_out(a)

            wait_gather(t, a)
            pbase = lax.rem(t * CHUNK, SEQ_LEN)

            def add_pos(k, c):
                l = pbase + k
                l = lax.select(l >= SEQ_LEN, l - SEQ_LEN, l)
                ph = (l & 1) * EMBED_DIM
                lh = l // 2
                for q in range(EMBED_DIM // LANES):
                    sl = pl.ds(q * LANES, LANES)
                    rows[a][k, sl] = rows[a][k, sl] + pos_v[lh, pl.ds(ph + q * LANES, LANES)]
                return c

            lax.fori_loop(0, CHUNK, add_pos, 0, unroll=4)
            pltpu.async_copy(
                rows[a], out_hbm.at[pl.ds(base + t * CHUNK, CHUNK)], so[a]
            )

            tn = t + 2

            @pl.when(tn < NCHUNK)
            def _():
                fire_gather(tn, (a + 2) % NBUF)

        return carry

    lax.fori_loop(0, NCHUNK // NBUF, outer, 0)

    for a in range(NBUF):
        wait_out(a)


def kernel(x, token_emb_table):
    pos2 = _pos_encoding().reshape(SEQ_LEN // 2, 128)
    table_p = jnp.pad(token_emb_table, ((0, 0), (0, 128 - EMBED_DIM)))
    x_r = x.astype(jnp.int32).reshape(NW, NCHUNK, CHUNK)
    out = _emb_kernel(x_r, table_p, pos2)
    return out[:, :EMBED_DIM].reshape(BATCH, SEQ_LEN, EMBED_DIM)
